# Initial kernel scaffold; baseline (speedup 1.0000x reference)
import jax
import jax.numpy as jnp
from jax.experimental import pallas as pl


def kernel(masks, noise):
    raise NotImplementedError("write your pallas kernel here")


# speedup vs baseline: 11.5232x; 11.5232x over previous
import functools

import jax
import jax.numpy as jnp
from jax import lax
from jax.experimental import pallas as pl
from jax.experimental.pallas import tpu as pltpu
from jax.experimental.pallas import tpu_sc as plsc

B = 64
N = 32768
NP = N // 4
NCHUNK = N // 16
NB = 1024
LANES = 16
BIG = 0x3FFFFFFF
CPAD = 128
IMAX = 0x7FFFFFFF


def _sc_body(nb_hbm, mp_hbm, out_hbm,
             keys_v, out_v, mp_v, hist_v, histm_v, cbuf_v, call_v, counts_sh):
    c = lax.axis_index("c")
    s = lax.axis_index("s")
    w = c * 16 + s

    iota = lax.iota(jnp.int32, LANES)
    iota_div4 = iota >> 2
    shifts8 = (iota & 3) << 3
    ones = jnp.full((LANES,), 1, jnp.int32)
    zeros = jnp.zeros((LANES,), jnp.int32)

    def count_row(j, cvec):
        row = s * 4 + j
        pltpu.sync_copy(mp_hbm.at[row], mp_v)

        def cbody(i, acc):
            for u in range(4):
                x = mp_v[pl.ds((i * 4 + u) * LANES, LANES)]
                acc = acc + ((x * jnp.int32(0x01010101)) >> 24)
            return acc

        acc = lax.fori_loop(0, NP // (4 * LANES), cbody, zeros)
        cnt = jnp.sum(acc)
        return jnp.where(iota == j, cnt, cvec)

    cvec = lax.fori_loop(0, 4, count_row, jnp.full((LANES,), BIG, jnp.int32))
    for j in range(0, CPAD, LANES):
        cbuf_v[pl.ds(j, LANES)] = cvec
    pltpu.sync_copy(cbuf_v, counts_sh.at[s])
    plsc.subcore_barrier()
    pltpu.sync_copy(counts_sh, call_v)

    macc = call_v[0, pl.ds(0, LANES)]
    for j in range(1, 16):
        macc = jnp.minimum(macc, call_v[j, pl.ds(0, LANES)])
    M = jnp.min(macc)
    Mc = jnp.maximum(M, 1)

    def radix_level(valid_of, id_of, C, M_rem):
        def hbody(i, _):
            for u in range(4):
                ch = i * 4 + u
                k = keys_v[pl.ds(ch * LANES, LANES)]
                valid = valid_of(k)
                off = (iota << 10) + id_of(k)
                plsc.addupdate_scatter(hist_v, [off], ones, mask=valid)
            return 0

        lax.fori_loop(0, NCHUNK // 4, hbody, 0)

        def mbody(i, _):
            acc = hist_v[pl.ds(i * LANES, LANES)]
            hist_v[pl.ds(i * LANES, LANES)] = zeros
            for l in range(1, 16):
                sl = pl.ds(l * NB + i * LANES, LANES)
                acc = acc + hist_v[sl]
                hist_v[sl] = zeros
            histm_v[pl.ds(i * LANES, LANES)] = acc
            return 0

        lax.fori_loop(0, NB // LANES, mbody, 0)

        thresh = C - M_rem

        def sbody(i, carry):
            run, cv = carry
            h = histm_v[pl.ds(i * LANES, LANES)]
            pc = plsc.cumsum(h) + run
            cv = cv + jnp.where(pc <= thresh, 1, 0)
            return jnp.max(pc), cv

        _, cv = lax.fori_loop(0, NB // LANES, sbody, (jnp.int32(0), zeros))
        t = jnp.sum(cv)

        def abody(i, acc):
            h = histm_v[pl.ds(i * LANES, LANES)]
            return acc + jnp.where(iota + i * LANES > t, h, 0)

        S_t1 = jnp.sum(lax.fori_loop(0, NB // LANES, abody, zeros))
        C_next = jnp.max(plsc.load_gather(histm_v, [iota * 0 + t]))
        M_next = jnp.maximum(1, M_rem - S_t1)
        return t, C_next, M_next

    def do_row(j, _):
        row = w * 2 + j
        pltpu.sync_copy(nb_hbm.at[row], keys_v)
        pltpu.sync_copy(mp_hbm.at[row], mp_v)
        crow = call_v[row >> 2, pl.ds(0, LANES)]
        C0 = jnp.sum(jnp.where(iota == (row & 3), crow, 0))

        def zbody(i, _):
            for u in range(4):
                hist_v[pl.ds((i * 4 + u) * LANES, LANES)] = zeros
            return 0

        lax.fori_loop(0, (16 * NB) // (4 * LANES), zbody, 0)

        def p0body(i, _):
            for u in range(4):
                ch = i * 4 + u
                nb = keys_v[pl.ds(ch * LANES, LANES)]
                g = plsc.load_gather(mp_v, [iota_div4 + ch * 4])
                valid = ((g >> shifts8) & 1) == 1
                k = jnp.where(valid, nb, -1)
                keys_v[pl.ds(ch * LANES, LANES)] = k
                off = (iota << 10) + (k >> 20)
                plsc.addupdate_scatter(hist_v, [off], ones, mask=valid)
            return 0

        lax.fori_loop(0, NCHUNK // 4, p0body, 0)

        def mbody(i, _):
            acc = hist_v[pl.ds(i * LANES, LANES)]
            hist_v[pl.ds(i * LANES, LANES)] = zeros
            for l in range(1, 16):
                sl = pl.ds(l * NB + i * LANES, LANES)
                acc = acc + hist_v[sl]
                hist_v[sl] = zeros
            histm_v[pl.ds(i * LANES, LANES)] = acc
            return 0

        lax.fori_loop(0, NB // LANES, mbody, 0)
        thresh = C0 - Mc

        def sbody(i, carry):
            run, cv = carry
            h = histm_v[pl.ds(i * LANES, LANES)]
            pc = plsc.cumsum(h) + run
            cv = cv + jnp.where(pc <= thresh, 1, 0)
            return jnp.max(pc), cv

        _, cv = lax.fori_loop(0, NB // LANES, sbody, (jnp.int32(0), zeros))
        t0 = jnp.sum(cv)

        def abody(i, acc):
            h = histm_v[pl.ds(i * LANES, LANES)]
            return acc + jnp.where(iota + i * LANES > t0, h, 0)

        S_t1 = jnp.sum(lax.fori_loop(0, NB // LANES, abody, zeros))
        C1 = jnp.max(plsc.load_gather(histm_v, [iota * 0 + t0]))
        M1 = jnp.maximum(1, Mc - S_t1)

        t1, C2, M2 = radix_level(
            lambda k: (k >> 20) == t0,
            lambda k: (k >> 10) & (NB - 1),
            C1, M1)
        pref1 = t0 * NB + t1
        t2, C3, M3 = radix_level(
            lambda k: (k >> 10) == pref1,
            lambda k: k & (NB - 1),
            C2, M2)
        T = pref1 * NB + t2
        T_eff = jnp.where(M == 0, IMAX, T)
        no_tie = jnp.logical_or(M3 >= C3, M == 0)

        @pl.when(no_tie)
        def _fast():
            def fbody(i, _):
                for u in range(4):
                    ch = i * 4 + u
                    k = keys_v[pl.ds(ch * LANES, LANES)]
                    out_v[pl.ds(ch * LANES, LANES)] = jnp.where(k >= T_eff, 1, 0)
                return 0

            lax.fori_loop(0, NCHUNK // 4, fbody, 0)

        @pl.when(jnp.logical_not(no_tie))
        def _tie():
            def tbody(i, run):
                k = keys_v[pl.ds(i * LANES, LANES)]
                eq = k == T
                pe = plsc.cumsum(jnp.where(eq, 1, 0)) + run
                keep = (k > T) | (eq & (pe <= M3))
                out_v[pl.ds(i * LANES, LANES)] = jnp.where(keep, 1, 0)
                return jnp.max(pe)

            lax.fori_loop(0, NCHUNK, tbody, jnp.int32(0))

        pltpu.sync_copy(out_v, out_hbm.at[row])
        return 0

    lax.fori_loop(0, 2, do_row, 0)


@functools.partial(
    pl.kernel,
    out_type=jax.ShapeDtypeStruct((B, N), jnp.int32),
    mesh=plsc.VectorSubcoreMesh(core_axis_name="c", subcore_axis_name="s",
                                num_cores=2, num_subcores=16),
    compiler_params=pltpu.CompilerParams(needs_layout_passes=False),
    scratch_types=[
        pltpu.VMEM((N,), jnp.int32),
        pltpu.VMEM((N,), jnp.int32),
        pltpu.VMEM((NP,), jnp.int32),
        pltpu.VMEM((16 * NB,), jnp.int32),
        pltpu.VMEM((NB,), jnp.int32),
        pltpu.VMEM((CPAD,), jnp.int32),
        pltpu.VMEM((16, CPAD), jnp.int32),
        pltpu.VMEM_SHARED((16, CPAD), jnp.int32),
    ],
)
def _rect_sc(nb_hbm, mp_hbm, out_hbm,
             keys_v, out_v, mp_v, hist_v, histm_v, cbuf_v, call_v, counts_sh):
    _sc_body(nb_hbm, mp_hbm, out_hbm,
             keys_v, out_v, mp_v, hist_v, histm_v, cbuf_v, call_v, counts_sh)


def kernel(masks, noise):
    shape = masks.shape
    m = masks.reshape(B, N)
    nb = lax.bitcast_convert_type(noise.reshape(B, N), jnp.int32)
    mp = lax.bitcast_convert_type(m.astype(jnp.int8).reshape(B, NP, 4),
                                  jnp.int32)
    out = _rect_sc(nb, mp)
    return out.astype(jnp.bool_).reshape(shape)


# speedup vs baseline: 19.0821x; 1.6560x over previous
import functools

import jax
import jax.numpy as jnp
from jax import lax
from jax.experimental import pallas as pl
from jax.experimental.pallas import tpu as pltpu
from jax.experimental.pallas import tpu_sc as plsc

B = 64
N = 32768
NP = N // 4
NCHUNK = N // 16
NB = 1024
LANES = 16
BIG = 0x3FFFFFFF
CPAD = 128
IMAX = 0x7FFFFFFF


def _sc_body(nb_hbm, mp_hbm, out_hbm,
             keys_v, out_v, mp_v, hist_v, cbuf_v, call_v, counts_sh):
    c = lax.axis_index("c")
    s = lax.axis_index("s")
    w = c * 16 + s

    iota = lax.iota(jnp.int32, LANES)
    ones = jnp.full((LANES,), 1, jnp.int32)
    zeros = jnp.zeros((LANES,), jnp.int32)
    negones = jnp.full((LANES,), -1, jnp.int32)

    def count_row(j, cvec):
        row = s * 4 + j
        pltpu.sync_copy(mp_hbm.at[row], mp_v)

        @functools.partial(plsc.parallel_loop, 0, NP // LANES, unroll=8,
                           carry=zeros)
        def acc(i, a):
            x = mp_v[pl.ds(i * LANES, LANES)]
            return a + ((x * jnp.int32(0x01010101)) >> 24)

        cnt = jnp.sum(acc)
        return jnp.where(iota == j, cnt, cvec)

    cvec = lax.fori_loop(0, 4, count_row, jnp.full((LANES,), BIG, jnp.int32))
    for j in range(0, CPAD, LANES):
        cbuf_v[pl.ds(j, LANES)] = cvec
    pltpu.sync_copy(cbuf_v, counts_sh.at[s])
    plsc.subcore_barrier()
    pltpu.sync_copy(counts_sh, call_v)

    macc = call_v[0, pl.ds(0, LANES)]
    for j in range(1, 16):
        macc = jnp.minimum(macc, call_v[j, pl.ds(0, LANES)])
    M = jnp.min(macc)
    Mc = jnp.maximum(M, 1)

    for j in range(0, NB, LANES):
        hist_v[pl.ds(j, LANES)] = zeros

    def rank_walk(C, M_rem):
        thresh = C - M_rem

        def sbody(i, carry):
            run, cv = carry
            h = hist_v[pl.ds(i * LANES, LANES)]
            pc = plsc.cumsum(h) + run
            cv = cv + jnp.where(pc <= thresh, 1, 0)
            return jnp.max(pc), cv

        _, cv = lax.fori_loop(0, NB // LANES, sbody, (jnp.int32(0), zeros))
        t = jnp.sum(cv)
        C_next = jnp.max(plsc.load_gather(hist_v, [iota * 0 + t]))

        def abody(i, acc):
            sl = pl.ds(i * LANES, LANES)
            h = hist_v[sl]
            hist_v[sl] = zeros
            return acc + jnp.where(iota + i * LANES > t, h, 0)

        S_t1 = jnp.sum(lax.fori_loop(0, NB // LANES, abody, zeros))
        M_next = jnp.maximum(1, M_rem - S_t1)
        return t, C_next, M_next

    def hist_level(valid_of, id_of):
        @functools.partial(plsc.parallel_loop, 0, NCHUNK, unroll=8)
        def _(i):
            k = keys_v[pl.ds(i * LANES, LANES)]
            plsc.addupdate_scatter(hist_v, [id_of(k)], ones, mask=valid_of(k))

    def do_row(j, _):
        row = w * 2 + j
        pltpu.sync_copy(nb_hbm.at[row], keys_v)
        pltpu.sync_copy(mp_hbm.at[row], mp_v)
        crow = call_v[row >> 2, pl.ds(0, LANES)]
        C0 = jnp.sum(jnp.where(iota == (row & 3), crow, 0))

        @functools.partial(plsc.parallel_loop, 0, NCHUNK, unroll=8)
        def _(i):
            nb = keys_v[pl.ds(i * LANES, LANES)]
            g = plsc.load_gather(mp_v, [iota_div4 + i * 4])
            valid = ((g >> shifts8) & 1) == 1
            k = jnp.where(valid, nb, -1)
            keys_v[pl.ds(i * LANES, LANES)] = k
            plsc.addupdate_scatter(hist_v, [k >> 20], ones, mask=valid)

        t0, C1, M1 = rank_walk(C0, Mc)
        hist_level(lambda k: (k >> 20) == t0, lambda k: (k >> 10) & (NB - 1))
        t1, C2, M2 = rank_walk(C1, M1)
        pref1 = t0 * NB + t1
        hist_level(lambda k: (k >> 10) == pref1, lambda k: k & (NB - 1))
        t2, C3, M3 = rank_walk(C2, M2)
        T = pref1 * NB + t2
        T_eff = jnp.where(M == 0, IMAX, T)
        no_tie = jnp.logical_or(M3 >= C3, M == 0)

        @pl.when(no_tie)
        def _fast():
            @functools.partial(plsc.parallel_loop, 0, NCHUNK, unroll=8)
            def _(i):
                k = keys_v[pl.ds(i * LANES, LANES)]
                out_v[pl.ds(i * LANES, LANES)] = jnp.where(k >= T_eff, 1, 0)

        @pl.when(jnp.logical_not(no_tie))
        def _tie():
            def tbody(i, run):
                k = keys_v[pl.ds(i * LANES, LANES)]
                eq = k == T
                pe = plsc.cumsum(jnp.where(eq, 1, 0)) + run
                keep = (k > T) | (eq & (pe <= M3))
                out_v[pl.ds(i * LANES, LANES)] = jnp.where(keep, 1, 0)
                return jnp.max(pe)

            lax.fori_loop(0, NCHUNK, tbody, jnp.int32(0))

        pltpu.sync_copy(out_v, out_hbm.at[pl.ds(row * N, N)])
        return 0

    lax.fori_loop(0, 2, do_row, 0)


@functools.partial(
    pl.kernel,
    out_type=jax.ShapeDtypeStruct((B * N,), jnp.int8),
    mesh=plsc.VectorSubcoreMesh(core_axis_name="c", subcore_axis_name="s",
                                num_cores=2, num_subcores=16),
    compiler_params=pltpu.CompilerParams(needs_layout_passes=False),
    scratch_types=[
        pltpu.VMEM((N,), jnp.int32),
        pltpu.VMEM((N,), jnp.int8),
        pltpu.VMEM((NP,), jnp.int32),
        pltpu.VMEM((NB,), jnp.int32),
        pltpu.VMEM((CPAD,), jnp.int32),
        pltpu.VMEM((16, CPAD), jnp.int32),
        pltpu.VMEM_SHARED((16, CPAD), jnp.int32),
    ],
)
def _rect_sc(nb_hbm, mp_hbm, out_hbm,
             keys_v, out_v, mp_v, hist_v, cbuf_v, call_v, counts_sh):
    _sc_body(nb_hbm, mp_hbm, out_hbm,
             keys_v, out_v, mp_v, hist_v, cbuf_v, call_v, counts_sh)


def kernel(masks, noise):
    shape = masks.shape
    nb = lax.bitcast_convert_type(noise.reshape(B, N), jnp.int32)
    mp = lax.bitcast_convert_type(
        masks.reshape(B, N).astype(jnp.int8).reshape(B, NP, 4), jnp.int32)
    out = _rect_sc(nb, mp)
    return out.astype(jnp.bool_).reshape(shape)


# speedup vs baseline: 19.7790x; 1.0365x over previous
import functools

import jax
import jax.numpy as jnp
from jax import lax
from jax.experimental import pallas as pl
from jax.experimental.pallas import tpu as pltpu
from jax.experimental.pallas import tpu_sc as plsc

B = 64
N = 32768
NP = N // 4
NCHUNK = N // 16
NB = 1024
LANES = 16
BIG = 0x3FFFFFFF
CPAD = 128
IMAX = 0x7FFFFFFF


def _sc_body(nb_hbm, mp_hbm, out_hbm,
             keys_v, out_v, mp_v, hist_v, cbuf_v, call_v, counts_sh):
    c = lax.axis_index("c")
    s = lax.axis_index("s")
    w = c * 16 + s

    iota = lax.iota(jnp.int32, LANES)
    iota_div4 = iota >> 2
    shifts8 = (iota & 3) << 3
    ones = jnp.full((LANES,), 1, jnp.int32)
    zeros = jnp.zeros((LANES,), jnp.int32)

    def count_row(j, cvec):
        row = s * 4 + j
        pltpu.sync_copy(mp_hbm.at[row], mp_v)

        @plsc.parallel_loop(0, NP // LANES, unroll=8, carry=zeros)
        def acc(i, a):
            x = mp_v[pl.ds(i * LANES, LANES)]
            return a + ((x * jnp.int32(0x01010101)) >> 24)

        cnt = jnp.sum(acc)
        return jnp.where(iota == j, cnt, cvec)

    cvec = lax.fori_loop(0, 4, count_row, jnp.full((LANES,), BIG, jnp.int32))
    for j in range(0, CPAD, LANES):
        cbuf_v[pl.ds(j, LANES)] = cvec
    pltpu.sync_copy(cbuf_v, counts_sh.at[s])
    plsc.subcore_barrier()
    pltpu.sync_copy(counts_sh, call_v)

    macc = call_v[0, pl.ds(0, LANES)]
    for j in range(1, 16):
        macc = jnp.minimum(macc, call_v[j, pl.ds(0, LANES)])
    M = jnp.min(macc)
    Mc = jnp.maximum(M, 1)

    for j in range(0, NB, LANES):
        hist_v[pl.ds(j, LANES)] = zeros

    def rank_walk(C, M_rem):
        thresh = C - M_rem

        def sbody(i, carry):
            run, cv = carry
            h = hist_v[pl.ds(i * LANES, LANES)]
            pc = plsc.cumsum(h) + run
            cv = cv + jnp.where(pc <= thresh, 1, 0)
            return jnp.max(pc), cv

        _, cv = lax.fori_loop(0, NB // LANES, sbody, (jnp.int32(0), zeros))
        t = jnp.sum(cv)
        C_next = jnp.max(plsc.load_gather(hist_v, [iota * 0 + t]))

        def abody(i, acc):
            sl = pl.ds(i * LANES, LANES)
            h = hist_v[sl]
            hist_v[sl] = zeros
            return acc + jnp.where(iota + i * LANES > t, h, 0)

        S_t1 = jnp.sum(lax.fori_loop(0, NB // LANES, abody, zeros))
        M_next = jnp.maximum(1, M_rem - S_t1)
        return t, C_next, M_next

    def hist_level(valid_of, id_of):
        @plsc.parallel_loop(0, NCHUNK, unroll=16)
        def _(i):
            k = keys_v[pl.ds(i * LANES, LANES)]
            plsc.addupdate_scatter(hist_v, [id_of(k)], ones, mask=valid_of(k))

    def do_row(j, _):
        row = w * 2 + j
        pltpu.sync_copy(nb_hbm.at[row], keys_v)
        pltpu.sync_copy(mp_hbm.at[row], mp_v)
        crow = call_v[row >> 2, pl.ds(0, LANES)]
        C0 = jnp.sum(jnp.where(iota == (row & 3), crow, 0))

        @plsc.parallel_loop(0, NCHUNK, unroll=16)
        def _(i):
            nb = keys_v[pl.ds(i * LANES, LANES)]
            g = plsc.load_gather(mp_v, [iota_div4 + i * 4])
            valid = ((g >> shifts8) & 1) == 1
            k = jnp.where(valid, nb, -1)
            keys_v[pl.ds(i * LANES, LANES)] = k
            plsc.addupdate_scatter(hist_v, [k >> 20], ones, mask=valid)

        t0, C1, M1 = rank_walk(C0, Mc)
        hist_level(lambda k: (k >> 20) == t0, lambda k: (k >> 10) & (NB - 1))
        t1, C2, M2 = rank_walk(C1, M1)
        pref1 = t0 * NB + t1
        hist_level(lambda k: (k >> 10) == pref1, lambda k: k & (NB - 1))
        t2, C3, M3 = rank_walk(C2, M2)
        T = pref1 * NB + t2
        T_eff = jnp.where(M == 0, IMAX, T)
        no_tie = jnp.logical_or(M3 >= C3, M == 0)

        @pl.when(no_tie)
        def _fast():
            @plsc.parallel_loop(0, NCHUNK, unroll=16)
            def _(i):
                k = keys_v[pl.ds(i * LANES, LANES)]
                out_v[pl.ds(i * LANES, LANES)] = jnp.where(k >= T_eff, 1, 0)

        @pl.when(jnp.logical_not(no_tie))
        def _tie():
            def tbody(i, run):
                k = keys_v[pl.ds(i * LANES, LANES)]
                eq = k == T
                pe = plsc.cumsum(jnp.where(eq, 1, 0)) + run
                keep = (k > T) | (eq & (pe <= M3))
                out_v[pl.ds(i * LANES, LANES)] = jnp.where(keep, 1, 0)
                return jnp.max(pe)

            lax.fori_loop(0, NCHUNK, tbody, jnp.int32(0))

        pltpu.sync_copy(out_v, out_hbm.at[row])
        return 0

    lax.fori_loop(0, 2, do_row, 0)


@functools.partial(
    pl.kernel,
    out_type=jax.ShapeDtypeStruct((B, N), jnp.int32),
    mesh=plsc.VectorSubcoreMesh(core_axis_name="c", subcore_axis_name="s",
                                num_cores=2, num_subcores=16),
    compiler_params=pltpu.CompilerParams(needs_layout_passes=False),
    scratch_types=[
        pltpu.VMEM((N,), jnp.int32),
        pltpu.VMEM((N,), jnp.int32),
        pltpu.VMEM((NP,), jnp.int32),
        pltpu.VMEM((NB,), jnp.int32),
        pltpu.VMEM((CPAD,), jnp.int32),
        pltpu.VMEM((16, CPAD), jnp.int32),
        pltpu.VMEM_SHARED((16, CPAD), jnp.int32),
    ],
)
def _rect_sc(nb_hbm, mp_hbm, out_hbm,
             keys_v, out_v, mp_v, hist_v, cbuf_v, call_v, counts_sh):
    _sc_body(nb_hbm, mp_hbm, out_hbm,
             keys_v, out_v, mp_v, hist_v, cbuf_v, call_v, counts_sh)


def kernel(masks, noise):
    shape = masks.shape
    m = masks.reshape(B, N)
    nb = lax.bitcast_convert_type(noise.reshape(B, N), jnp.int32)
    mp = lax.bitcast_convert_type(m.astype(jnp.int8).reshape(B, NP, 4),
                                  jnp.int32)
    out = _rect_sc(nb, mp)
    return out.astype(jnp.bool_).reshape(shape)


# speedup vs baseline: 27.6394x; 1.3974x over previous
import functools

import jax
import jax.numpy as jnp
from jax import lax
from jax.experimental import pallas as pl
from jax.experimental.pallas import tpu as pltpu
from jax.experimental.pallas import tpu_sc as plsc

B = 64
N = 32768
NP = N // 4
NCHUNK = N // 16
NB = 1024
LANES = 16
BIG = 0x3FFFFFFF
CPAD = 128
IMAX = 0x7FFFFFFF


def _sc_body(nb_hbm, mp_hbm, out_hbm,
             keys_v, out_v, mp_v, hist_v, cbuf_v, call_v, counts_sh):
    c = lax.axis_index("c")
    s = lax.axis_index("s")
    w = c * 16 + s

    iota = lax.iota(jnp.int32, LANES)
    iota_div4 = iota >> 2
    shifts8 = (iota & 3) << 3
    ones = jnp.full((LANES,), 1, jnp.int32)
    zeros = jnp.zeros((LANES,), jnp.int32)

    def count_row(j, cvec):
        row = s * 4 + j
        pltpu.sync_copy(mp_hbm.at[row], mp_v)

        @plsc.parallel_loop(0, NP // LANES, unroll=8, carry=zeros)
        def acc(i, a):
            x = mp_v[pl.ds(i * LANES, LANES)]
            return a + ((x * jnp.int32(0x01010101)) >> 24)

        cnt = jnp.sum(acc)
        return jnp.where(iota == j, cnt, cvec)

    cvec = lax.fori_loop(0, 4, count_row, jnp.full((LANES,), BIG, jnp.int32))
    for j in range(0, CPAD, LANES):
        cbuf_v[pl.ds(j, LANES)] = cvec
    pltpu.sync_copy(cbuf_v, counts_sh.at[s])
    plsc.subcore_barrier()
    pltpu.sync_copy(counts_sh, call_v)

    macc = call_v[0, pl.ds(0, LANES)]
    for j in range(1, 16):
        macc = jnp.minimum(macc, call_v[j, pl.ds(0, LANES)])
    M = jnp.min(macc)
    Mc = jnp.maximum(M, 1)

    for j in range(0, NB, LANES):
        hist_v[pl.ds(j, LANES)] = zeros

    def rank_walk(C, M_rem):
        thresh = C - M_rem

        def sbody(i, carry):
            run, cv = carry
            h = hist_v[pl.ds(i * LANES, LANES)]
            pc = plsc.cumsum(h) + run
            cv = cv + jnp.where(pc <= thresh, 1, 0)
            return jnp.max(pc), cv

        _, cv = lax.fori_loop(0, NB // LANES, sbody, (jnp.int32(0), zeros))
        t = jnp.sum(cv)
        C_next = jnp.max(plsc.load_gather(hist_v, [iota * 0 + t]))

        def abody(i, acc):
            sl = pl.ds(i * LANES, LANES)
            h = hist_v[sl]
            hist_v[sl] = zeros
            return acc + jnp.where(iota + i * LANES > t, h, 0)

        S_t1 = jnp.sum(lax.fori_loop(0, NB // LANES, abody, zeros))
        M_next = jnp.maximum(1, M_rem - S_t1)
        return t, C_next, M_next

    def hist_level(valid_of, id_of):
        @plsc.parallel_loop(0, NCHUNK, unroll=16)
        def _(i):
            k = keys_v[pl.ds(i * LANES, LANES)]
            plsc.addupdate_scatter(hist_v, [id_of(k)], ones, mask=valid_of(k))

    def do_row(j, _):
        row = w * 2 + j
        pltpu.sync_copy(nb_hbm.at[row], keys_v)
        pltpu.sync_copy(mp_hbm.at[row], mp_v)
        crow = call_v[row >> 2, pl.ds(0, LANES)]
        C0 = jnp.sum(jnp.where(iota == (row & 3), crow, 0))

        @plsc.parallel_loop(0, NCHUNK, unroll=16)
        def _(i):
            nb = keys_v[pl.ds(i * LANES, LANES)]
            g = plsc.load_gather(mp_v, [iota_div4 + i * 4])
            valid = ((g >> shifts8) & 1) == 1
            k = jnp.where(valid, nb, -1)
            keys_v[pl.ds(i * LANES, LANES)] = k
            plsc.addupdate_scatter(hist_v, [k >> 20], ones, mask=valid)

        t0, C1, M1 = rank_walk(C0, Mc)
        hist_level(lambda k: (k >> 20) == t0, lambda k: (k >> 10) & (NB - 1))
        t1, C2, M2 = rank_walk(C1, M1)
        pref1 = t0 * NB + t1
        hist_level(lambda k: (k >> 10) == pref1, lambda k: k & (NB - 1))
        t2, C3, M3 = rank_walk(C2, M2)
        T = pref1 * NB + t2
        T_eff = jnp.where(M == 0, IMAX, T)
        no_tie = jnp.logical_or(M3 >= C3, M == 0)

        @pl.when(no_tie)
        def _fast():
            @plsc.parallel_loop(0, NCHUNK, unroll=16)
            def _(i):
                k = keys_v[pl.ds(i * LANES, LANES)]
                out_v[pl.ds(i * LANES, LANES)] = jnp.where(k >= T_eff, 1, 0)

        @pl.when(jnp.logical_not(no_tie))
        def _tie():
            def tbody(i, run):
                k = keys_v[pl.ds(i * LANES, LANES)]
                eq = k == T
                pe = plsc.cumsum(jnp.where(eq, 1, 0)) + run
                keep = (k > T) | (eq & (pe <= M3))
                out_v[pl.ds(i * LANES, LANES)] = jnp.where(keep, 1, 0)
                return jnp.max(pe)

            lax.fori_loop(0, NCHUNK, tbody, jnp.int32(0))

        pltpu.sync_copy(out_v, out_hbm.at[row])
        return 0

    lax.fori_loop(0, 2, do_row, 0)


@functools.partial(
    pl.kernel,
    out_type=jax.ShapeDtypeStruct((B, N), jnp.int32),
    mesh=plsc.VectorSubcoreMesh(core_axis_name="c", subcore_axis_name="s",
                                num_cores=2, num_subcores=16),
    compiler_params=pltpu.CompilerParams(needs_layout_passes=False),
    scratch_types=[
        pltpu.VMEM((N,), jnp.int32),
        pltpu.VMEM((N,), jnp.int32),
        pltpu.VMEM((NP,), jnp.int32),
        pltpu.VMEM((NB,), jnp.int32),
        pltpu.VMEM((CPAD,), jnp.int32),
        pltpu.VMEM((16, CPAD), jnp.int32),
        pltpu.VMEM_SHARED((16, CPAD), jnp.int32),
    ],
)
def _rect_sc(nb_hbm, mp_hbm, out_hbm,
             keys_v, out_v, mp_v, hist_v, cbuf_v, call_v, counts_sh):
    _sc_body(nb_hbm, mp_hbm, out_hbm,
             keys_v, out_v, mp_v, hist_v, cbuf_v, call_v, counts_sh)


def kernel(masks, noise):
    shape = masks.shape
    m = masks.reshape(B, N)
    nb = lax.bitcast_convert_type(noise.reshape(B, N), jnp.int32)
    weights = jnp.array([1.0, 256.0, 65536.0, 16777216.0], jnp.float32)
    mp = jnp.dot(m.reshape(B, NP, 4).astype(jnp.float32),
                 weights).astype(jnp.int32)
    out = _rect_sc(nb, mp)
    return out.astype(jnp.bool_).reshape(shape)
